# baseline (speedup 1.0000x reference)
import functools

import jax
import jax.numpy as jnp
from jax import lax
from jax.experimental import pallas as pl
from jax.experimental.pallas import tpu as pltpu
from jax.experimental.pallas import tpu_sc as plsc

B = 16384
F = 26
V = 100000
D = 64
NW = 10
OUT_D = 32

NC = 2
NS = 16
NWORK = NC * NS

FB = F * B
ROWS_PER_W = FB // NWORK
CHUNK = 128
NCHUNK = ROWS_PER_W // CHUNK


def _leaky(x):
    return jnp.where(x >= 0, x, 0.2 * x)


def _sc_gather_body(table_hbm, idx_hbm, out_hbm, idx_v, rows0, rows1, sem0, sem1):
    wid = lax.axis_index("s") * NC + lax.axis_index("c")
    base = wid * ROWS_PER_W
    pltpu.sync_copy(idx_hbm.at[wid], idx_v)

    def start(c, rows, sem):
        pltpu.async_copy(table_hbm.at[idx_v.at[c]], rows, sem)

    def wait(rows, sem):
        pltpu.make_async_copy(table_hbm.at[idx_v.at[0]], rows, sem).wait()

    def write(c, rows):
        pltpu.sync_copy(rows, out_hbm.at[pl.ds(base + c * CHUNK, CHUNK)])

    start(0, rows0, sem0)
    start(1, rows1, sem1)

    def body(g, carry):
        c = 2 * g
        wait(rows0, sem0)
        write(c, rows0)
        start(c + 2, rows0, sem0)
        wait(rows1, sem1)
        write(c + 1, rows1)
        start(c + 3, rows1, sem1)
        return carry

    lax.fori_loop(0, NCHUNK // 2 - 1, body, 0, unroll=False)
    c_last = NCHUNK - 2
    wait(rows0, sem0)
    write(c_last, rows0)
    wait(rows1, sem1)
    write(c_last + 1, rows1)


@jax.jit
def _sc_gather(tables_flat, idx3):
    mesh = plsc.VectorSubcoreMesh(core_axis_name="c", subcore_axis_name="s")
    return pl.kernel(
        _sc_gather_body,
        out_type=jax.ShapeDtypeStruct((FB, D), jnp.float32),
        mesh=mesh,
        scratch_types=[
            pltpu.VMEM((NCHUNK, CHUNK), jnp.int32),
            pltpu.VMEM((CHUNK, D), jnp.float32),
            pltpu.VMEM((CHUNK, D), jnp.float32),
            pltpu.SemaphoreType.DMA,
            pltpu.SemaphoreType.DMA,
        ],
        compiler_params=pltpu.CompilerParams(use_tc_tiling_on_sc=False),
    )(tables_flat, idx3)


def _wide_prep_body(wide_ref, gamma_ref, beta_ref, ww_ref, bw_ref, wp_ref, bp_ref):
    x = wide_ref[...]
    mean = jnp.mean(x, axis=1, keepdims=True)
    var = jnp.mean((x - mean) ** 2, axis=1, keepdims=True)
    scale = gamma_ref[...].T * lax.rsqrt(var + 1e-5)
    shift = beta_ref[...].T - mean * scale
    wp_ref[...] = scale * ww_ref[...]
    bp_ref[...] = jnp.sum(shift * ww_ref[...], axis=0, keepdims=True) + bw_ref[...]


@jax.jit
def _wide_prep(wide_in, gamma, beta, Ww, bw):
    return pl.pallas_call(
        _wide_prep_body,
        out_shape=[
            jax.ShapeDtypeStruct((NW, OUT_D), jnp.float32),
            jax.ShapeDtypeStruct((1, OUT_D), jnp.float32),
        ],
    )(wide_in, gamma.reshape(1, NW), beta.reshape(1, NW), Ww, bw.reshape(1, OUT_D))


BB = 512


def _main_body(emb_ref, wd_ref, bd_ref, wide_ref, wp_ref, bp_ref, out_ref, acc_ref):
    f = pl.program_id(1)

    @pl.when(f == 0)
    def _init():
        acc_ref[...] = jnp.zeros_like(acc_ref)

    acc_ref[...] += jnp.dot(
        emb_ref[0], wd_ref[0], preferred_element_type=jnp.float32
    )

    @pl.when(f == F - 1)
    def _final():
        deep = _leaky(acc_ref[...] + bd_ref[...])
        wide = lax.dot_general(
            wide_ref[...], wp_ref[...], (((0,), (0,)), ((), ())),
            preferred_element_type=jnp.float32,
        )
        wide = _leaky(wide + bp_ref[...])
        out_ref[...] = jnp.concatenate([deep, wide], axis=1)


@jax.jit
def _main(emb, Wd3, bd2, wide_in, Wp, bp):
    grid = (B // BB, F)
    return pl.pallas_call(
        _main_body,
        grid=grid,
        in_specs=[
            pl.BlockSpec((1, BB, D), lambda i, f: (f, i, 0)),
            pl.BlockSpec((1, D, OUT_D), lambda i, f: (f, 0, 0)),
            pl.BlockSpec((1, OUT_D), lambda i, f: (0, 0)),
            pl.BlockSpec((NW, BB), lambda i, f: (0, i)),
            pl.BlockSpec((NW, OUT_D), lambda i, f: (0, 0)),
            pl.BlockSpec((1, OUT_D), lambda i, f: (0, 0)),
        ],
        out_specs=pl.BlockSpec((BB, 2 * OUT_D), lambda i, f: (i, 0)),
        out_shape=jax.ShapeDtypeStruct((B, 2 * OUT_D), jnp.float32),
        scratch_shapes=[pltpu.VMEM((BB, OUT_D), jnp.float32)],
        compiler_params=pltpu.CompilerParams(
            dimension_semantics=("parallel", "arbitrary"),
        ),
    )(emb, Wd3, bd2, wide_in, Wp, bp)


def kernel(deep_in, wide_in, tables, Wd, bd, gamma, beta, Ww, bw):
    offs = (jnp.arange(F, dtype=jnp.int32) * V)[:, None]
    flat_idx = (deep_in.astype(jnp.int32) + offs).reshape(NWORK, NCHUNK, CHUNK)
    tables_flat = tables.reshape(F * V, D)

    emb_flat = _sc_gather(tables_flat, flat_idx)
    Wp, bp = _wide_prep(wide_in, gamma, beta, Ww, bw)
    out = _main(
        emb_flat.reshape(F, B, D),
        Wd.reshape(F, D, OUT_D),
        bd.reshape(1, OUT_D),
        wide_in,
        Wp,
        bp,
    )
    return out
